# Initial kernel scaffold; baseline (speedup 1.0000x reference)
import jax
import jax.numpy as jnp
from jax.experimental import pallas as pl


def kernel(obs, l1_Wq, l1_Wk, l1_Wv, l1_Wo, l1_bo, l2_Wq, l2_Wk, l2_Wv, l2_Wo, l2_bo, mlp_W1, mlp_b1, mlp_W2, mlp_b2, mlp_W3, mlp_b3):
    raise NotImplementedError("write your pallas kernel here")


# speedup vs baseline: 539.9173x; 539.9173x over previous
import functools

import jax
import jax.numpy as jnp
from jax.experimental import pallas as pl

B, A, D_IN = 128, 32, 128
POS_DIM = 2
HID = 64
HEADS = 4
N_OUT = 2
MLP_H = 256

GB = 8
BLK = GB * A
SCALE = float(HID // HEADS) ** -0.5


def _gat_attention(pos, x_in, wq, wk, wv, wo, bo, valid):
    q = jnp.dot(pos, wq, preferred_element_type=jnp.float32)
    k = jnp.dot(pos, wk, preferred_element_type=jnp.float32)
    v = jnp.dot(x_in, wv, preferred_element_type=jnp.float32)
    outs = []
    for h in range(HEADS):
        sl = slice(h * HID, (h + 1) * HID)
        s = jnp.dot(q[:, sl], k[:, sl].T,
                    preferred_element_type=jnp.float32) * SCALE
        s = jnp.where(valid, s, -jnp.inf)
        m = jnp.max(s, axis=1, keepdims=True)
        e = jnp.exp(s - m)
        den = jnp.sum(e, axis=1, keepdims=True)
        attn = e / (den + 1e-16)
        outs.append(jnp.dot(attn, v[:, sl], preferred_element_type=jnp.float32))
    o = jnp.concatenate(outs, axis=1)
    return jnp.dot(o, wo, preferred_element_type=jnp.float32) + bo


def _fused_body(x_ref, wq1, wk1, wv1, wo1, bo1, wq2, wk2, wv2, wo2, bo2,
                w1, b1, w2, b2, w3, b3, o_ref):
    x = x_ref[:, :]
    pos = x[:, :POS_DIM]

    row = jax.lax.broadcasted_iota(jnp.int32, (BLK, BLK), 0)
    col = jax.lax.broadcasted_iota(jnp.int32, (BLK, BLK), 1)
    valid = ((row // A) == (col // A)) & (row != col)

    h = _gat_attention(pos, x, wq1[:, :], wk1[:, :], wv1[:, :], wo1[:, :],
                       bo1[:, :], valid)
    h = jnp.tanh(h)
    h = _gat_attention(pos, h, wq2[:, :], wk2[:, :], wv2[:, :], wo2[:, :],
                       bo2[:, :], valid)
    h = jnp.tanh(h)

    h = jnp.maximum(jnp.dot(h, w1[:, :], preferred_element_type=jnp.float32)
                    + b1[:, :], 0.0)
    h = jnp.maximum(jnp.dot(h, w2[:, :], preferred_element_type=jnp.float32)
                    + b2[:, :], 0.0)
    o_ref[:, :] = (jnp.dot(h, w3[:, :], preferred_element_type=jnp.float32)
                   + b3[:, :])


def _full(shape):
    return pl.BlockSpec(shape, lambda i: (0, 0))


@jax.jit
def kernel(obs, l1_Wq, l1_Wk, l1_Wv, l1_Wo, l1_bo, l2_Wq, l2_Wk, l2_Wv,
           l2_Wo, l2_bo, mlp_W1, mlp_b1, mlp_W2, mlp_b2, mlp_W3, mlp_b3):
    N = B * A
    x = obs.reshape(N, D_IN)
    hc = HEADS * HID
    grid = N // BLK

    out = pl.pallas_call(
        _fused_body,
        grid=(grid,),
        in_specs=[
            pl.BlockSpec((BLK, D_IN), lambda i: (i, 0)),
            _full((POS_DIM, hc)), _full((POS_DIM, hc)),
            _full((D_IN, hc)), _full((hc, HID)), _full((1, HID)),
            _full((POS_DIM, hc)), _full((POS_DIM, hc)),
            _full((HID, hc)), _full((hc, HID)), _full((1, HID)),
            _full((HID, MLP_H)), _full((1, MLP_H)),
            _full((MLP_H, MLP_H)), _full((1, MLP_H)),
            _full((MLP_H, N_OUT)), _full((1, N_OUT)),
        ],
        out_specs=pl.BlockSpec((BLK, N_OUT), lambda i: (i, 0)),
        out_shape=jax.ShapeDtypeStruct((N, N_OUT), jnp.float32),
    )(x, l1_Wq, l1_Wk, l1_Wv, l1_Wo, l1_bo.reshape(1, HID),
      l2_Wq, l2_Wk, l2_Wv, l2_Wo, l2_bo.reshape(1, HID),
      mlp_W1, mlp_b1.reshape(1, MLP_H), mlp_W2, mlp_b2.reshape(1, MLP_H),
      mlp_W3, mlp_b3.reshape(1, N_OUT))
    return out.reshape(B, A, N_OUT)
